# Initial kernel scaffold; baseline (speedup 1.0000x reference)
import jax
import jax.numpy as jnp
from jax.experimental import pallas as pl


def kernel(hidden, rel_emb, query, selected_edges, m1_proj_W, m1_proj_b, m1_gate_W, m1_gate_b, m1_cond_W, m1_out_W, m2_proj_W, m2_proj_b, m2_gate_W, m2_gate_b, m2_out_W, h1_proj_W, h1_proj_b, h1_gate_W, h1_gate_b, h1_cond_W, h1_out_W, h2_proj_W, h2_proj_b, h2_gate_W, h2_gate_b, h2_out_W):
    raise NotImplementedError("write your pallas kernel here")


# speedup vs baseline: 2.5585x; 2.5585x over previous
import functools

import jax
import jax.numpy as jnp
from jax import lax
from jax.experimental import pallas as pl
from jax.experimental.pallas import tpu as pltpu
from jax.experimental.pallas import tpu_sc as plsc

N = 10000
E = 160000
D = 256

EB = 256
W = EB + 8
NPAD = 10256
NB = 1000

SC_NC = 2
SC_NS = 16
SC_W = SC_NC * SC_NS
SC_C = 128
SC_CHUNKS = E // SC_C
SC_ITERS = -(-SC_CHUNKS // SC_W)


def _gather_body(hidden_hbm, src_hbm, out_hbm, idx_v, rows_v, sem):
    wid = lax.axis_index("s") * SC_NC + lax.axis_index("c")

    def step(i, carry):
        chunk = wid + i * SC_W

        @pl.when(chunk < SC_CHUNKS)
        def _():
            off = chunk * SC_C
            pltpu.sync_copy(src_hbm.at[pl.ds(off, SC_C)], idx_v)
            pltpu.async_copy(hidden_hbm.at[idx_v], rows_v, sem).wait()
            pltpu.sync_copy(rows_v, out_hbm.at[pl.ds(off, SC_C)])

        return carry

    lax.fori_loop(0, SC_ITERS, step, 0)


def _sc_gather(hidden, src, *, interpret=False):
    mesh = plsc.VectorSubcoreMesh(core_axis_name="c", subcore_axis_name="s")
    f = pl.kernel(
        _gather_body,
        out_type=jax.ShapeDtypeStruct((E, D), jnp.float32),
        mesh=mesh,
        scratch_types=[
            pltpu.VMEM((SC_C,), jnp.int32),
            pltpu.VMEM((SC_C, D), jnp.float32),
            pltpu.SemaphoreType.DMA,
        ],
        interpret=interpret,
    )
    return f(hidden, src)


def _edge_body(bases_s, hv_ref, re_ref, dst_ref,
               m1pW, m1pb, m1gW, m1gb, m1cW, m1oW,
               m2pW, m2pb, m2gW, m2gb, m2oW, q_ref,
               acc_sum, acc_cnt):
    b = pl.program_id(0)
    base = bases_s[b]
    base_al = (base // 8) * 8

    hv = hv_ref[...]
    re = re_ref[...]

    m1p = m1pW[...]
    m1g = m1gW[...]
    p1 = (jnp.dot(hv, m1p[:D], preferred_element_type=jnp.float32)
          + jnp.dot(re, m1p[D:], preferred_element_type=jnp.float32)
          + m1pb[...])
    g1 = (jnp.dot(hv, m1g[:D], preferred_element_type=jnp.float32)
          + jnp.dot(re, m1g[D:], preferred_element_type=jnp.float32)
          + m1gb[...])
    cond = jnp.tanh(jnp.dot(q_ref[...], m1cW[...],
                            preferred_element_type=jnp.float32))
    o1 = jnp.tanh(p1) * jax.nn.sigmoid(g1) * cond
    f1 = jnp.tanh(jnp.dot(o1, m1oW[...], preferred_element_type=jnp.float32))

    m2p = m2pW[...]
    m2g = m2gW[...]
    p2 = (jnp.dot(hv, m2p[:D], preferred_element_type=jnp.float32)
          + jnp.dot(re, m2p[D:2 * D], preferred_element_type=jnp.float32)
          + jnp.dot(f1, m2p[2 * D:], preferred_element_type=jnp.float32)
          + m2pb[...])
    g2 = (jnp.dot(hv, m2g[:D], preferred_element_type=jnp.float32)
          + jnp.dot(re, m2g[D:2 * D], preferred_element_type=jnp.float32)
          + jnp.dot(f1, m2g[2 * D:], preferred_element_type=jnp.float32)
          + m2gb[...])
    o2 = jnp.tanh(p2) * jax.nn.sigmoid(g2)
    msg = jnp.tanh(jnp.dot(o2, m2oW[...], preferred_element_type=jnp.float32))

    col = dst_ref[0] - base_al
    rows = lax.broadcasted_iota(jnp.int32, (W, EB), 0)
    selT = (rows == col).astype(jnp.float32)
    part_sum = jnp.dot(selT, msg, preferred_element_type=jnp.float32)
    part_cnt = jnp.dot(selT, jnp.ones((EB, 128), jnp.float32),
                       preferred_element_type=jnp.float32)

    @pl.when(b == 0)
    def _():
        acc_sum[...] = jnp.zeros_like(acc_sum)
        acc_cnt[...] = jnp.zeros_like(acc_cnt)

    acc_sum[pl.ds(base_al, W), :] = acc_sum[pl.ds(base_al, W), :] + part_sum
    acc_cnt[pl.ds(base_al, W), :] = acc_cnt[pl.ds(base_al, W), :] + part_cnt


def _edge_call(hidden_vi, rel_emb, dst3, bases,
               m1pW, m1pb, m1gW, m1gb, m1cW, m1oW,
               m2pW, m2pb, m2gW, m2gb, m2oW, query, *, interpret=False):
    nblk = E // EB
    rep = lambda shape: pl.BlockSpec(shape, lambda b, bases: (0,) * len(shape))
    grid_spec = pltpu.PrefetchScalarGridSpec(
        num_scalar_prefetch=1,
        grid=(nblk,),
        in_specs=[
            pl.BlockSpec((EB, D), lambda b, bases: (b, 0)),
            pl.BlockSpec((EB, D), lambda b, bases: (b, 0)),
            pl.BlockSpec((1, 1, EB), lambda b, bases: (b, 0, 0)),
            rep((2 * D, D)), rep((1, D)), rep((2 * D, D)), rep((1, D)),
            rep((D, D)), rep((D, D)),
            rep((3 * D, D)), rep((1, D)), rep((3 * D, D)), rep((1, D)),
            rep((D, D)), rep((1, D)),
        ],
        out_specs=[
            pl.BlockSpec((NPAD, D), lambda b, bases: (0, 0)),
            pl.BlockSpec((NPAD, 128), lambda b, bases: (0, 0)),
        ],
    )
    return pl.pallas_call(
        _edge_body,
        grid_spec=grid_spec,
        out_shape=[
            jax.ShapeDtypeStruct((NPAD, D), jnp.float32),
            jax.ShapeDtypeStruct((NPAD, 128), jnp.float32),
        ],
        interpret=interpret,
    )(bases, hidden_vi, rel_emb, dst3,
      m1pW, m1pb, m1gW, m1gb, m1cW, m1oW,
      m2pW, m2pb, m2gW, m2gb, m2oW, query)


def _node_body(hid_ref, sum_ref, cnt_ref,
               h1pW, h1pb, h1gW, h1gb, h1cW, h1oW,
               h2pW, h2pb, h2gW, h2gb, h2oW, q_ref, out_ref):
    hid = hid_ref[...]
    cnt = cnt_ref[:, 0:1]
    aggr = sum_ref[...] / jnp.maximum(cnt, 1.0)

    h1p = h1pW[...]
    h1g = h1gW[...]
    p1 = (jnp.dot(hid, h1p[:D], preferred_element_type=jnp.float32)
          + jnp.dot(aggr, h1p[D:], preferred_element_type=jnp.float32)
          + h1pb[...])
    g1 = (jnp.dot(hid, h1g[:D], preferred_element_type=jnp.float32)
          + jnp.dot(aggr, h1g[D:], preferred_element_type=jnp.float32)
          + h1gb[...])
    cond = jnp.tanh(jnp.dot(q_ref[...], h1cW[...],
                            preferred_element_type=jnp.float32))
    o1 = jnp.tanh(p1) * jax.nn.sigmoid(g1) * cond
    f1 = jnp.tanh(jnp.dot(o1, h1oW[...], preferred_element_type=jnp.float32))

    h2p = h2pW[...]
    h2g = h2gW[...]
    p2 = (jnp.dot(hid, h2p[:D], preferred_element_type=jnp.float32)
          + jnp.dot(aggr, h2p[D:2 * D], preferred_element_type=jnp.float32)
          + jnp.dot(f1, h2p[2 * D:], preferred_element_type=jnp.float32)
          + h2pb[...])
    g2 = (jnp.dot(hid, h2g[:D], preferred_element_type=jnp.float32)
          + jnp.dot(aggr, h2g[D:2 * D], preferred_element_type=jnp.float32)
          + jnp.dot(f1, h2g[2 * D:], preferred_element_type=jnp.float32)
          + h2gb[...])
    o2 = jnp.tanh(p2) * jax.nn.sigmoid(g2)
    out_ref[...] = hid + jnp.tanh(
        jnp.dot(o2, h2oW[...], preferred_element_type=jnp.float32))


def _node_call(hidden, acc_sum, acc_cnt,
               h1pW, h1pb, h1gW, h1gb, h1cW, h1oW,
               h2pW, h2pb, h2gW, h2gb, h2oW, query, *, interpret=False):
    nblk = N // NB
    rep = lambda shape: pl.BlockSpec(shape, lambda b: (0,) * len(shape))
    return pl.pallas_call(
        _node_body,
        grid=(nblk,),
        in_specs=[
            pl.BlockSpec((NB, D), lambda b: (b, 0)),
            pl.BlockSpec((NB, D), lambda b: (b, 0)),
            pl.BlockSpec((NB, 128), lambda b: (b, 0)),
            rep((2 * D, D)), rep((1, D)), rep((2 * D, D)), rep((1, D)),
            rep((D, D)), rep((D, D)),
            rep((3 * D, D)), rep((1, D)), rep((3 * D, D)), rep((1, D)),
            rep((D, D)), rep((1, D)),
        ],
        out_specs=pl.BlockSpec((NB, D), lambda b: (b, 0)),
        out_shape=jax.ShapeDtypeStruct((N, D), jnp.float32),
        interpret=interpret,
    )(hidden, acc_sum, acc_cnt,
      h1pW, h1pb, h1gW, h1gb, h1cW, h1oW,
      h2pW, h2pb, h2gW, h2gb, h2oW, query)


def kernel(hidden, rel_emb, query, selected_edges,
           m1_proj_W, m1_proj_b, m1_gate_W, m1_gate_b, m1_cond_W, m1_out_W,
           m2_proj_W, m2_proj_b, m2_gate_W, m2_gate_b, m2_out_W,
           h1_proj_W, h1_proj_b, h1_gate_W, h1_gate_b, h1_cond_W, h1_out_W,
           h2_proj_W, h2_proj_b, h2_gate_W, h2_gate_b, h2_out_W):
    src = selected_edges[:, 6]
    dst = selected_edges[:, 5]
    dst3 = dst.reshape(E // EB, 1, EB)
    bases = dst[::EB]

    hidden_vi = _sc_gather(hidden, src)

    acc_sum, acc_cnt = _edge_call(
        hidden_vi, rel_emb, dst3, bases,
        m1_proj_W, m1_proj_b.reshape(1, D), m1_gate_W, m1_gate_b.reshape(1, D),
        m1_cond_W, m1_out_W,
        m2_proj_W, m2_proj_b.reshape(1, D), m2_gate_W, m2_gate_b.reshape(1, D),
        m2_out_W, query)

    return _node_call(
        hidden, acc_sum, acc_cnt,
        h1_proj_W, h1_proj_b.reshape(1, D), h1_gate_W, h1_gate_b.reshape(1, D),
        h1_cond_W, h1_out_W,
        h2_proj_W, h2_proj_b.reshape(1, D), h2_gate_W, h2_gate_b.reshape(1, D),
        h2_out_W, query)
